# baseline (speedup 1.0000x reference)
import functools

import jax
import jax.numpy as jnp
from jax.experimental import pallas as pl
from jax.experimental.pallas import tpu as pltpu


def _partial_kernel(b_ref, xl_ref, xr_ref, o_ref, c_ref, *, num_graphs, hf):
    ni = pl.program_id(1)

    @pl.when(ni == 0)
    def _init():
        o_ref[...] = jnp.zeros_like(o_ref)
        c_ref[...] = jnp.zeros_like(c_ref)

    b = b_ref[...]
    gids = jax.lax.broadcasted_iota(jnp.int32, (num_graphs, b.shape[1]), 0)
    m = (gids == b).astype(jnp.float32)
    c_ref[...] += jnp.sum(m, axis=1, keepdims=True)[None]
    o_ref[0, :, :hf] += jnp.dot(m, xl_ref[...],
                                preferred_element_type=jnp.float32)
    o_ref[0, :, hf:] += jnp.dot(m, xr_ref[...],
                                preferred_element_type=jnp.float32)


def _combine_kernel(p_ref, c_ref, o_ref):
    c = c_ref[0] + c_ref[1]
    p = p_ref[0] + p_ref[1]
    o_ref[...] = p / jnp.maximum(c, 1.0)


def _reduce_mean(node_feat, batch, num_graphs, tile_n=4096, tile_f=128):
    n, f = node_feat.shape
    num_n = n // tile_n
    half = num_n // 2

    hf = f // 2
    b2 = batch.astype(jnp.int32).reshape(1, n)
    partial, cnt = pl.pallas_call(
        functools.partial(_partial_kernel, num_graphs=num_graphs, hf=hf),
        out_shape=(jax.ShapeDtypeStruct((2, num_graphs, f), jnp.float32),
                   jax.ShapeDtypeStruct((2, num_graphs, 1), jnp.float32)),
        grid=(2, half),
        in_specs=[
            pl.BlockSpec((1, tile_n), lambda ci, ni: (0, ci * half + ni)),
            pl.BlockSpec((tile_n, hf), lambda ci, ni: (ci * half + ni, 0)),
            pl.BlockSpec((tile_n, hf), lambda ci, ni: (ci * half + ni, 1)),
        ],
        out_specs=(pl.BlockSpec((1, num_graphs, f), lambda ci, ni: (ci, 0, 0)),
                   pl.BlockSpec((1, num_graphs, 1), lambda ci, ni: (ci, 0, 0))),
        compiler_params=pltpu.CompilerParams(
            dimension_semantics=("parallel", "arbitrary")),
    )(b2, node_feat, node_feat)

    return pl.pallas_call(
        _combine_kernel,
        out_shape=jax.ShapeDtypeStruct((num_graphs, f), jnp.float32),
        grid=(f // tile_f,),
        in_specs=[
            pl.BlockSpec((2, num_graphs, tile_f), lambda fi: (0, 0, fi)),
            pl.BlockSpec((2, num_graphs, 1), lambda fi: (0, 0, 0)),
        ],
        out_specs=pl.BlockSpec((num_graphs, tile_f), lambda fi: (0, fi)),
        compiler_params=pltpu.CompilerParams(
            dimension_semantics=("parallel",)),
    )(partial, cnt)


def kernel(node_feat, batch):
    return _reduce_mean(jnp.asarray(node_feat), jnp.asarray(batch), 512)


# speedup vs baseline: 1.4456x; 1.4456x over previous
import functools

import jax
import jax.numpy as jnp
from jax.experimental import pallas as pl
from jax.experimental.pallas import tpu as pltpu


def _partial_kernel(b_ref, x_ref, o_ref, c_ref, *, num_graphs):
    ni = pl.program_id(1)

    @pl.when(ni == 0)
    def _init():
        o_ref[...] = jnp.zeros_like(o_ref)
        c_ref[...] = jnp.zeros_like(c_ref)

    b = b_ref[...]
    gids = jax.lax.broadcasted_iota(jnp.int32, (num_graphs, b.shape[1]), 0)
    m = (gids == b).astype(jnp.bfloat16)
    c_ref[...] += jnp.sum(m, axis=1, keepdims=True,
                          dtype=jnp.float32)[None]
    x = x_ref[...].astype(jnp.bfloat16)
    o_ref[...] += jnp.dot(m, x,
                          preferred_element_type=jnp.float32)[None]


def _combine_kernel(p_ref, c_ref, o_ref):
    c = c_ref[0] + c_ref[1]
    p = p_ref[0] + p_ref[1]
    o_ref[...] = p / jnp.maximum(c, 1.0)


def _reduce_mean(node_feat, batch, num_graphs, tile_n=4096, tile_f=128):
    n, f = node_feat.shape
    num_n = n // tile_n
    half = num_n // 2

    b2 = batch.astype(jnp.int32).reshape(1, n)
    partial, cnt = pl.pallas_call(
        functools.partial(_partial_kernel, num_graphs=num_graphs),
        out_shape=(jax.ShapeDtypeStruct((2, num_graphs, f), jnp.float32),
                   jax.ShapeDtypeStruct((2, num_graphs, 1), jnp.float32)),
        grid=(2, half),
        in_specs=[
            pl.BlockSpec((1, tile_n), lambda ci, ni: (0, ci * half + ni)),
            pl.BlockSpec((tile_n, f), lambda ci, ni: (ci * half + ni, 0)),
        ],
        out_specs=(pl.BlockSpec((1, num_graphs, f), lambda ci, ni: (ci, 0, 0)),
                   pl.BlockSpec((1, num_graphs, 1), lambda ci, ni: (ci, 0, 0))),
        compiler_params=pltpu.CompilerParams(
            dimension_semantics=("parallel", "arbitrary")),
    )(b2, node_feat)

    return pl.pallas_call(
        _combine_kernel,
        out_shape=jax.ShapeDtypeStruct((num_graphs, f), jnp.float32),
        grid=(f // tile_f,),
        in_specs=[
            pl.BlockSpec((2, num_graphs, tile_f), lambda fi: (0, 0, fi)),
            pl.BlockSpec((2, num_graphs, 1), lambda fi: (0, 0, 0)),
        ],
        out_specs=pl.BlockSpec((num_graphs, tile_f), lambda fi: (0, fi)),
        compiler_params=pltpu.CompilerParams(
            dimension_semantics=("parallel",)),
    )(partial, cnt)


def kernel(node_feat, batch):
    return _reduce_mean(jnp.asarray(node_feat), jnp.asarray(batch), 512)


# speedup vs baseline: 1.5739x; 1.0888x over previous
import functools

import jax
import jax.numpy as jnp
from jax.experimental import pallas as pl
from jax.experimental.pallas import tpu as pltpu


def _partial_kernel(b_ref, x_ref, o_ref, c_ref, *, num_graphs):
    ni = pl.program_id(1)

    @pl.when(ni == 0)
    def _init():
        o_ref[...] = jnp.zeros_like(o_ref)
        c_ref[...] = jnp.zeros_like(c_ref)

    b = b_ref[...]
    gids = jax.lax.broadcasted_iota(jnp.int32, (num_graphs, b.shape[1]), 0)
    m = (gids == b).astype(jnp.bfloat16)
    c_ref[...] += jnp.sum(m, axis=1, keepdims=True,
                          dtype=jnp.float32)[None]
    x = x_ref[...].astype(jnp.bfloat16)
    o_ref[...] += jnp.dot(m, x,
                          preferred_element_type=jnp.float32)[None]


def _combine_kernel(p_ref, c_ref, o_ref):
    c = c_ref[0] + c_ref[1]
    p = p_ref[0] + p_ref[1]
    o_ref[...] = p / jnp.maximum(c, 1.0)


def _reduce_mean(node_feat, batch, num_graphs, tile_n=8192, tile_f=128):
    n, f = node_feat.shape
    num_n = n // tile_n
    half = num_n // 2

    b2 = batch.astype(jnp.int32).reshape(1, n)
    partial, cnt = pl.pallas_call(
        functools.partial(_partial_kernel, num_graphs=num_graphs),
        out_shape=(jax.ShapeDtypeStruct((2, num_graphs, f), jnp.float32),
                   jax.ShapeDtypeStruct((2, num_graphs, 1), jnp.float32)),
        grid=(2, half),
        in_specs=[
            pl.BlockSpec((1, tile_n), lambda ci, ni: (0, ci * half + ni)),
            pl.BlockSpec((tile_n, f), lambda ci, ni: (ci * half + ni, 0)),
        ],
        out_specs=(pl.BlockSpec((1, num_graphs, f), lambda ci, ni: (ci, 0, 0)),
                   pl.BlockSpec((1, num_graphs, 1), lambda ci, ni: (ci, 0, 0))),
        compiler_params=pltpu.CompilerParams(
            dimension_semantics=("parallel", "arbitrary")),
    )(b2, node_feat)

    return pl.pallas_call(
        _combine_kernel,
        out_shape=jax.ShapeDtypeStruct((num_graphs, f), jnp.float32),
        grid=(f // tile_f,),
        in_specs=[
            pl.BlockSpec((2, num_graphs, tile_f), lambda fi: (0, 0, fi)),
            pl.BlockSpec((2, num_graphs, 1), lambda fi: (0, 0, 0)),
        ],
        out_specs=pl.BlockSpec((num_graphs, tile_f), lambda fi: (0, fi)),
        compiler_params=pltpu.CompilerParams(
            dimension_semantics=("parallel",)),
    )(partial, cnt)


def kernel(node_feat, batch):
    return _reduce_mean(jnp.asarray(node_feat), jnp.asarray(batch), 512)
